# Initial kernel scaffold; baseline (speedup 1.0000x reference)
import jax
import jax.numpy as jnp
from jax.experimental import pallas as pl


def kernel(x, Wr, w1, w2):
    raise NotImplementedError("write your pallas kernel here")


# speedup vs baseline: 4.6849x; 4.6849x over previous
import functools

import jax
import jax.numpy as jnp
from jax.experimental import pallas as pl
from jax.experimental.pallas import tpu as pltpu

T = 128
D_MODEL = 1024
D_FF = 1024
E = 64


def _moe_body(x_ref, wr_ref, w1_ref, w2_ref, out_ref, g_ref):
    e = pl.program_id(0)

    @pl.when(e == 0)
    def _init():
        logits = jnp.dot(x_ref[...], wr_ref[...],
                         preferred_element_type=jnp.float32)
        m = jnp.max(logits, axis=-1, keepdims=True)
        ex = jnp.exp(logits - m)
        s = jnp.sum(ex, axis=-1, keepdims=True)
        eidx = jnp.argmax(logits, axis=-1)
        lane = jax.lax.broadcasted_iota(jnp.int32, (T, E), 1)
        onehot = lane == eidx[:, None]
        g_ref[...] = jnp.where(onehot, ex / s, 0.0)
        out_ref[...] = jnp.zeros_like(out_ref)

    lane = jax.lax.broadcasted_iota(jnp.int32, (T, E), 1)
    g_e = jnp.sum(jnp.where(lane == e, g_ref[...], 0.0),
                  axis=1, keepdims=True)
    any_hit = jnp.sum(g_e) > 0.0

    @pl.when(any_hit)
    def _compute():
        h = jnp.dot(x_ref[...], w1_ref[0],
                    preferred_element_type=jnp.float32)
        h = h * jax.nn.sigmoid(h)
        h = h * g_e
        out_ref[...] += jnp.dot(h, w2_ref[0],
                                preferred_element_type=jnp.float32)


@jax.jit
def kernel(x, Wr, w1, w2):
    return pl.pallas_call(
        _moe_body,
        grid=(E,),
        in_specs=[
            pl.BlockSpec((T, D_MODEL), lambda e: (0, 0)),
            pl.BlockSpec((D_MODEL, E), lambda e: (0, 0)),
            pl.BlockSpec((1, D_MODEL, D_FF), lambda e: (e, 0, 0)),
            pl.BlockSpec((1, D_FF, D_MODEL), lambda e: (e, 0, 0)),
        ],
        out_specs=pl.BlockSpec((T, D_MODEL), lambda e: (0, 0)),
        out_shape=jax.ShapeDtypeStruct((T, D_MODEL), jnp.float32),
        scratch_shapes=[pltpu.VMEM((T, E), jnp.float32)],
        compiler_params=pltpu.CompilerParams(
            dimension_semantics=("arbitrary",),
        ),
    )(x, Wr, w1, w2)


# speedup vs baseline: 5.6249x; 1.2006x over previous
import jax
import jax.numpy as jnp
from jax.experimental import pallas as pl
from jax.experimental.pallas import tpu as pltpu

T = 128
D_MODEL = 1024
D_FF = 1024
E = 64


def _router_body(x_ref, wr_ref, g_ref, sched_ref, ucnt_ref):
    logits = jnp.dot(x_ref[...], wr_ref[...],
                     preferred_element_type=jnp.float32)
    m = jnp.max(logits, axis=-1, keepdims=True)
    ex = jnp.exp(logits - m)
    s = jnp.sum(ex, axis=-1, keepdims=True)
    eidx = jnp.argmax(logits, axis=-1)
    lane_te = jax.lax.broadcasted_iota(jnp.int32, (T, E), 1)
    onehot = lane_te == eidx[:, None]
    g_ref[...] = jnp.where(onehot, ex / s, 0.0)

    cnt = jnp.sum(onehot.astype(jnp.float32), axis=0)[None, :]
    used = cnt > 0.0
    used_f = used.astype(jnp.float32)
    iota_e = jax.lax.broadcasted_iota(jnp.int32, (1, E), 1)
    u_total = jnp.sum(used_f).astype(jnp.int32)
    last = jnp.max(jnp.where(used, iota_e, -1))

    rowe = jax.lax.broadcasted_iota(jnp.int32, (E, E), 0)
    colj = jax.lax.broadcasted_iota(jnp.int32, (E, E), 1)
    tri = (rowe <= colj).astype(jnp.float32)
    pos = jnp.dot(used_f, tri,
                  preferred_element_type=jnp.float32).astype(jnp.int32) - 1

    ident = rowe == colj
    pos_t = jnp.sum(jnp.where(ident, jnp.broadcast_to(pos, (E, E)), 0),
                    axis=1, keepdims=True)
    used_t = jnp.sum(jnp.where(ident,
                               jnp.broadcast_to(used_f, (E, E)), 0.0),
                     axis=1, keepdims=True) > 0.0
    sel = (pos_t == colj) & used_t
    schedv = jnp.sum(jnp.where(sel, rowe, 0), axis=0)[None, :]
    sched_ref[...] = jnp.where(iota_e < u_total, schedv, last)
    ucnt_ref[...] = jnp.full((1, 1), u_total, jnp.int32)


def _expert_body(sched_ref, ucnt_ref, x_ref, g_ref, w1_ref, w2_ref, out_ref):
    i = pl.program_id(0)

    @pl.when(i == 0)
    def _init():
        out_ref[...] = jnp.zeros_like(out_ref)

    @pl.when(i < ucnt_ref[0, 0])
    def _compute():
        e = sched_ref[0, i]
        lane = jax.lax.broadcasted_iota(jnp.int32, (T, E), 1)
        g_e = jnp.sum(jnp.where(lane == e, g_ref[...], 0.0),
                      axis=1, keepdims=True)
        h = jnp.dot(x_ref[...], w1_ref[0],
                    preferred_element_type=jnp.float32)
        h = h * jax.nn.sigmoid(h)
        h = h * g_e
        out_ref[...] += jnp.dot(h, w2_ref[0],
                                preferred_element_type=jnp.float32)


@jax.jit
def kernel(x, Wr, w1, w2):
    g, sched, ucnt = pl.pallas_call(
        _router_body,
        out_shape=(
            jax.ShapeDtypeStruct((T, E), jnp.float32),
            jax.ShapeDtypeStruct((1, E), jnp.int32),
            jax.ShapeDtypeStruct((1, 1), jnp.int32),
        ),
    )(x, Wr)

    return pl.pallas_call(
        _expert_body,
        grid_spec=pltpu.PrefetchScalarGridSpec(
            num_scalar_prefetch=2,
            grid=(E,),
            in_specs=[
                pl.BlockSpec((T, D_MODEL), lambda i, s, u: (0, 0)),
                pl.BlockSpec((T, E), lambda i, s, u: (0, 0)),
                pl.BlockSpec((1, D_MODEL, D_FF),
                             lambda i, s, u: (s[0, i], 0, 0)),
                pl.BlockSpec((1, D_FF, D_MODEL),
                             lambda i, s, u: (s[0, i], 0, 0)),
            ],
            out_specs=pl.BlockSpec((T, D_MODEL), lambda i, s, u: (0, 0)),
        ),
        out_shape=jax.ShapeDtypeStruct((T, D_MODEL), jnp.float32),
        compiler_params=pltpu.CompilerParams(
            dimension_semantics=("arbitrary",),
        ),
    )(sched, ucnt, x, g, w1, w2)
